# baseline (speedup 1.0000x reference)
import jax
import jax.numpy as jnp
from jax import lax
from jax.experimental import pallas as pl
from jax.experimental.pallas import tpu as pltpu

N = 10000
E = 320000
D = 128
S = 512
ITEMS = 100000

_f32 = jnp.float32
_i32 = jnp.int32
_DOT = dict(preferred_element_type=_f32, precision=lax.Precision.HIGHEST)


def _k2_fn(x_ref, w_ref, xs_ref, u_ref, v_ref):
    xw = lax.dot_general(x_ref[...], w_ref[...], (((1,), (0,)), ((), ())),
                         **_DOT)
    xs_ref[...] = xw[:, 0:D]
    u_ref[...] = xw[:, D:2 * D]
    v_ref[...] = xw[:, 2 * D:3 * D]


def _k2(x, wcat):
    bm = 1000
    return pl.pallas_call(
        _k2_fn,
        grid=(N // bm,),
        in_specs=[pl.BlockSpec((bm, D), lambda i: (i, 0)),
                  pl.BlockSpec((D, 3 * D), lambda i: (0, 0))],
        out_specs=[pl.BlockSpec((bm, D), lambda i: (i, 0))] * 3,
        out_shape=[jax.ShapeDtypeStruct((N, D), _f32)] * 3,
    )(x, wcat)


def _k2b_fn(ssum_ref, cnt_ref, wv_ref, bv_ref, sm_ref, fv_ref):
    sm = ssum_ref[...] / jnp.maximum(cnt_ref[...], 1.0)
    sm_ref[...] = sm
    fv_ref[...] = lax.dot_general(sm, wv_ref[...], (((1,), (0,)), ((), ())),
                                  **_DOT) + bv_ref[...]


def _k2b(ssum, cnt_bc, wv, bv2):
    return pl.pallas_call(
        _k2b_fn,
        out_shape=[jax.ShapeDtypeStruct((S, D), _f32)] * 2,
    )(ssum, cnt_bc, wv, bv2)


def _k5_fn(h_ref, w_ref, o_ref):
    o_ref[...] = lax.dot_general(h_ref[...], w_ref[...],
                                 (((1,), (0,)), ((), ())), **_DOT)


def _k5(h, wu):
    bm = 1000
    return pl.pallas_call(
        _k5_fn,
        grid=(N // bm,),
        in_specs=[pl.BlockSpec((bm, D), lambda i: (i, 0)),
                  pl.BlockSpec((D, D), lambda i: (0, 0))],
        out_specs=pl.BlockSpec((bm, D), lambda i: (i, 0)),
        out_shape=jax.ShapeDtypeStruct((N, D), _f32),
    )(h, wu)


_TI = 2048


def _k7_fn(rst_ref, den_ref, sm_ref, wsr_ref, emb_ref, out_ref, sr_scr):
    @pl.when(pl.program_id(0) == 0)
    def _():
        rstn = rst_ref[...] / jnp.maximum(den_ref[...], 1e-30)
        sr = (lax.dot_general(rstn, wsr_ref[0:D],
                              (((1,), (0,)), ((), ())), **_DOT)
              + lax.dot_general(sm_ref[...], wsr_ref[D:2 * D],
                                (((1,), (0,)), ((), ())), **_DOT))
        sr_scr[...] = sr
    out_ref[...] = lax.dot_general(sr_scr[...], emb_ref[...],
                                   (((1,), (1,)), ((), ())), **_DOT)


def _k7(rst_raw, den_bc, segmean, wsr, emb):
    ni = (ITEMS + _TI - 1) // _TI
    return pl.pallas_call(
        _k7_fn,
        grid=(ni,),
        in_specs=[pl.BlockSpec((S, D), lambda i: (0, 0)),
                  pl.BlockSpec((S, D), lambda i: (0, 0)),
                  pl.BlockSpec((S, D), lambda i: (0, 0)),
                  pl.BlockSpec((2 * D, D), lambda i: (0, 0)),
                  pl.BlockSpec((_TI, D), lambda i: (i, 0))],
        out_specs=pl.BlockSpec((S, _TI), lambda i: (0, i)),
        out_shape=jax.ShapeDtypeStruct((S, ITEMS), _f32),
        scratch_shapes=[pltpu.VMEM((S, D), _f32)],
    )(rst_raw, den_bc, segmean, wsr, emb)


def kernel(item_ids, edge_index, segment_ids, emb, W_self, W_neigh,
           W_self_r, W_neigh_r, Wu, Wv, bv, We, Wsr):
    src = edge_index[0]
    dst = edge_index[1]
    sid = segment_ids

    x = jnp.take(emb, item_ids, axis=0)

    wcat = jnp.concatenate([W_self + W_self_r, W_neigh, W_neigh_r], axis=1)
    xs, u, v = _k2(x, wcat)

    ones_e = jnp.ones((E,), _f32)
    degf = jax.ops.segment_sum(ones_e, dst, num_segments=N)
    degr = jax.ops.segment_sum(ones_e, src, num_segments=N)
    aggf = jax.ops.segment_sum(jnp.take(u, src, axis=0), dst,
                               num_segments=N) / jnp.clip(degf, 1.0)[:, None]
    aggr = jax.ops.segment_sum(jnp.take(v, dst, axis=0), src,
                               num_segments=N) / jnp.clip(degr, 1.0)[:, None]

    ssum = jax.ops.segment_sum(x, sid, num_segments=S)
    cnt = jax.ops.segment_sum(jnp.ones((N,), _f32), sid, num_segments=S)
    cnt_bc = jnp.broadcast_to(cnt[:, None], (S, D))
    segmean, featv = _k2b(ssum, cnt_bc, Wv, bv.reshape(1, D))

    h = jnp.maximum(xs + aggf + aggr, 0.0) + jnp.take(segmean, sid, axis=0)

    fu = _k5(h, Wu)

    sig = 1.0 / (1.0 + jnp.exp(-(fu + jnp.take(featv, sid, axis=0))))
    a = jnp.exp(sig @ We[:, 0])
    rst_raw = jax.ops.segment_sum(h * a[:, None], sid, num_segments=S)
    den = jax.ops.segment_sum(a, sid, num_segments=S)
    den_bc = jnp.broadcast_to(den[:, None], (S, D))

    return _k7(rst_raw, den_bc, segmean, Wsr, emb)


# speedup vs baseline: 2.6434x; 2.6434x over previous
import jax
import jax.numpy as jnp
from jax import lax
from jax.experimental import pallas as pl
from jax.experimental.pallas import tpu as pltpu

N = 10000
E = 320000
D = 128
S = 512
ITEMS = 100000

_f32 = jnp.float32
_i32 = jnp.int32
_DOT = dict(preferred_element_type=_f32, precision=lax.Precision.HIGHEST)

from jax.experimental.pallas import tpu_sc as plsc
import functools

NP = 10240
NC, NS = 2, 16
C = 80
EPT = E // NS
MAC = 2000
NMAC = EPT // MAC
NSEG = NP // NS

_mesh = plsc.VectorSubcoreMesh(core_axis_name="c", subcore_axis_name="s",
                               num_cores=NC, num_subcores=NS)


def _vcopy80(src1, base, dst):
    for i in range(5):
        dst[pl.ds(i * 16, 16)] = src1[pl.ds(base + i * 16, 16)]

@functools.partial(
    pl.kernel,
    out_type=(jax.ShapeDtypeStruct((NP, D), _f32),
              jax.ShapeDtypeStruct((NP, D), _f32)),
    mesh=_mesh,
    scratch_types=[pltpu.VMEM((MAC,), _i32),
                   pltpu.VMEM((MAC,), _i32),
                   pltpu.VMEM((C,), _i32),
                   pltpu.VMEM((C,), _i32),
                   pltpu.VMEM((C,), _i32),
                   pltpu.VMEM((C, D), _f32),
                   pltpu.VMEM((C, D), _f32),
                   pltpu.VMEM_SHARED((NP, D), _f32),
                   pltpu.SemaphoreType.DMA,
                   pltpu.SemaphoreType.DMA],
)
def _k3(src1, dst1, u, v, rdegf, rdegr, zb, aggf_out, aggr_out,
        gib_v, sib_v, gia_v, gib2_v, si_v, rba_v, rbb_v,
        agg_sp, sem0, sem1):
    c = lax.axis_index("c")
    t = lax.axis_index("s")

    pltpu.sync_copy(zb.at[pl.ds(t * NSEG, NSEG)],
                    agg_sp.at[pl.ds(t * NSEG, NSEG)])
    plsc.subcore_barrier()

    def direction(gidx1, sidx1, tab):
        gis = (gia_v, gib2_v)
        rbs = (rba_v, rbb_v)
        sems = (sem0, sem1)

        def macro(m, _):
            base = t * EPT + m * MAC
            pltpu.sync_copy(gidx1.at[pl.ds(base, MAC)], gib_v)
            pltpu.sync_copy(sidx1.at[pl.ds(base, MAC)], sib_v)
            _vcopy80(gib_v, 0, gia_v)
            _vcopy80(gib_v, C, gib2_v)
            descs = [pltpu.async_copy(tab.at[gia_v], rba_v, sem0),
                     pltpu.async_copy(tab.at[gib2_v], rbb_v, sem1)]
            for j in range(MAC // C):
                b = j % 2
                descs[b].wait()
                _vcopy80(sib_v, j * C, si_v)
                pltpu.sync_copy(rbs[b], agg_sp.at[si_v], add=True)
                if j + 2 < MAC // C:
                    _vcopy80(gib_v, (j + 2) * C, gis[b])
                    descs[b] = pltpu.async_copy(tab.at[gis[b]], rbs[b], sems[b])
            return 0

        lax.fori_loop(0, NMAC, macro, 0)

    @pl.when(c == 0)
    def _():
        direction(src1, dst1, u)

    @pl.when(c == 1)
    def _():
        direction(dst1, src1, v)

    plsc.subcore_barrier()

    def norm_chunk(q, rdeg, out):
        off = t * NSEG + q * C
        pltpu.sync_copy(agg_sp.at[pl.ds(off, C)], rba_v)
        pltpu.sync_copy(rdeg.at[pl.ds(off, C)], rbb_v)

        def body(r, _):
            for cc in range(D // 16):
                sl = pl.ds(cc * 16, 16)
                rba_v[r, sl] = rba_v[r, sl] * rbb_v[r, sl]
            return 0

        lax.fori_loop(0, C, body, 0)
        pltpu.sync_copy(rba_v, out.at[pl.ds(off, C)])

    @pl.when(c == 0)
    def _():
        for q in range(NSEG // C):
            norm_chunk(q, rdegf, aggf_out)

    @pl.when(c == 1)
    def _():
        for q in range(NSEG // C):
            norm_chunk(q, rdegr, aggr_out)


def _k2_fn(x_ref, w_ref, xs_ref, u_ref, v_ref):
    xw = lax.dot_general(x_ref[...], w_ref[...], (((1,), (0,)), ((), ())),
                         **_DOT)
    xs_ref[...] = xw[:, 0:D]
    u_ref[...] = xw[:, D:2 * D]
    v_ref[...] = xw[:, 2 * D:3 * D]


def _k2(x, wcat):
    bm = 1000
    return pl.pallas_call(
        _k2_fn,
        grid=(N // bm,),
        in_specs=[pl.BlockSpec((bm, D), lambda i: (i, 0)),
                  pl.BlockSpec((D, 3 * D), lambda i: (0, 0))],
        out_specs=[pl.BlockSpec((bm, D), lambda i: (i, 0))] * 3,
        out_shape=[jax.ShapeDtypeStruct((N, D), _f32)] * 3,
    )(x, wcat)


def _k2b_fn(ssum_ref, cnt_ref, wv_ref, bv_ref, sm_ref, fv_ref):
    sm = ssum_ref[...] / jnp.maximum(cnt_ref[...], 1.0)
    sm_ref[...] = sm
    fv_ref[...] = lax.dot_general(sm, wv_ref[...], (((1,), (0,)), ((), ())),
                                  **_DOT) + bv_ref[...]


def _k2b(ssum, cnt_bc, wv, bv2):
    return pl.pallas_call(
        _k2b_fn,
        out_shape=[jax.ShapeDtypeStruct((S, D), _f32)] * 2,
    )(ssum, cnt_bc, wv, bv2)


def _k5_fn(h_ref, w_ref, o_ref):
    o_ref[...] = lax.dot_general(h_ref[...], w_ref[...],
                                 (((1,), (0,)), ((), ())), **_DOT)


def _k5(h, wu):
    bm = 1000
    return pl.pallas_call(
        _k5_fn,
        grid=(N // bm,),
        in_specs=[pl.BlockSpec((bm, D), lambda i: (i, 0)),
                  pl.BlockSpec((D, D), lambda i: (0, 0))],
        out_specs=pl.BlockSpec((bm, D), lambda i: (i, 0)),
        out_shape=jax.ShapeDtypeStruct((N, D), _f32),
    )(h, wu)


_TI = 2048


def _k7_fn(rst_ref, den_ref, sm_ref, wsr_ref, emb_ref, out_ref, sr_scr):
    @pl.when(pl.program_id(0) == 0)
    def _():
        rstn = rst_ref[...] / jnp.maximum(den_ref[...], 1e-30)
        sr = (lax.dot_general(rstn, wsr_ref[0:D],
                              (((1,), (0,)), ((), ())), **_DOT)
              + lax.dot_general(sm_ref[...], wsr_ref[D:2 * D],
                                (((1,), (0,)), ((), ())), **_DOT))
        sr_scr[...] = sr
    out_ref[...] = lax.dot_general(sr_scr[...], emb_ref[...],
                                   (((1,), (1,)), ((), ())), **_DOT)


def _k7(rst_raw, den_bc, segmean, wsr, emb):
    ni = (ITEMS + _TI - 1) // _TI
    return pl.pallas_call(
        _k7_fn,
        grid=(ni,),
        in_specs=[pl.BlockSpec((S, D), lambda i: (0, 0)),
                  pl.BlockSpec((S, D), lambda i: (0, 0)),
                  pl.BlockSpec((S, D), lambda i: (0, 0)),
                  pl.BlockSpec((2 * D, D), lambda i: (0, 0)),
                  pl.BlockSpec((_TI, D), lambda i: (i, 0))],
        out_specs=pl.BlockSpec((S, _TI), lambda i: (0, i)),
        out_shape=jax.ShapeDtypeStruct((S, ITEMS), _f32),
        scratch_shapes=[pltpu.VMEM((S, D), _f32)],
    )(rst_raw, den_bc, segmean, wsr, emb)


def kernel(item_ids, edge_index, segment_ids, emb, W_self, W_neigh,
           W_self_r, W_neigh_r, Wu, Wv, bv, We, Wsr):
    src = edge_index[0]
    dst = edge_index[1]
    sid = segment_ids

    x = jnp.take(emb, item_ids, axis=0)

    wcat = jnp.concatenate([W_self + W_self_r, W_neigh, W_neigh_r], axis=1)
    xs, u, v = _k2(x, wcat)

    ones_e = jnp.ones((E,), _f32)
    degf = jax.ops.segment_sum(ones_e, dst, num_segments=N)
    degr = jax.ops.segment_sum(ones_e, src, num_segments=N)
    pad = jnp.ones((NP - N,), _f32)
    rdegf = jnp.broadcast_to(
        jnp.concatenate([1.0 / jnp.clip(degf, 1.0), pad])[:, None], (NP, D))
    rdegr = jnp.broadcast_to(
        jnp.concatenate([1.0 / jnp.clip(degr, 1.0), pad])[:, None], (NP, D))
    zb = jnp.zeros((NP, D), _f32)
    src1 = src.astype(_i32)
    dst1 = dst.astype(_i32)
    aggf_p, aggr_p = _k3(src1, dst1, u, v, rdegf, rdegr, zb)
    aggf = aggf_p[:N]
    aggr = aggr_p[:N]

    ssum = jax.ops.segment_sum(x, sid, num_segments=S)
    cnt = jax.ops.segment_sum(jnp.ones((N,), _f32), sid, num_segments=S)
    cnt_bc = jnp.broadcast_to(cnt[:, None], (S, D))
    segmean, featv = _k2b(ssum, cnt_bc, Wv, bv.reshape(1, D))

    h = jnp.maximum(xs + aggf + aggr, 0.0) + jnp.take(segmean, sid, axis=0)

    fu = _k5(h, Wu)

    sig = 1.0 / (1.0 + jnp.exp(-(fu + jnp.take(featv, sid, axis=0))))
    a = jnp.exp(sig @ We[:, 0])
    rst_raw = jax.ops.segment_sum(h * a[:, None], sid, num_segments=S)
    den = jax.ops.segment_sum(a, sid, num_segments=S)
    den_bc = jnp.broadcast_to(den[:, None], (S, D))

    return _k7(rst_raw, den_bc, segmean, Wsr, emb)
